# baseline (speedup 1.0000x reference)
import functools

import jax
import jax.numpy as jnp
from jax import lax
from jax.experimental import pallas as pl
from jax.experimental.pallas import tpu as pltpu
from jax.experimental.pallas import tpu_sc as plsc

L = 17
Q = 16
H_IN = 257
H_OUT = 256
B = 4
GROWS = B * H_OUT
NW = 32
ROWS_PER_W = GROWS // NW
IMG_COLS_PAD = 264
WQ_ROWS = 83968
OFFS = [((m >> 3) & 1) * (L * L * L) + ((m >> 2) & 1) * (L * L)
        + ((m >> 1) & 1) * L + (m & 1) for m in range(16)]

def _sc_out_type():
    return (jax.ShapeDtypeStruct((B * H_OUT * 4, H_OUT * 4), jnp.float32),
            jax.ShapeDtypeStruct((WQ_ROWS, 16), jnp.float32))


def _sc_scratch_types():
    return [
        pltpu.VMEM((ROWS_PER_W + 8, IMG_COLS_PAD), jnp.int32),
        pltpu.VMEM((2, 16, 128), jnp.int32),
        pltpu.VMEM((4, 256), jnp.float32),
        pltpu.VMEM((16, 128, 16), jnp.float32),
        pltpu.VMEM((16, 128, 16), jnp.float32),
        pltpu.VMEM((8, 1024), jnp.float32),
        pltpu.VMEM((1312, 16), jnp.float32),
        pltpu.SemaphoreType.DMA,
        pltpu.SemaphoreType.DMA,
    ]


def _sc_body(img_hbm, w_hbm, out_hbm, tbl_hbm,
               strip, idxb, frac, rows0, rows1, stag, stg,
               sem0, sem1):
    cid = lax.axis_index("c")
    sid = lax.axis_index("s")
    wid = sid * 2 + cid
    bidx = wid // (H_OUT // ROWS_PER_W)
    y0 = (wid % (H_OUT // ROWS_PER_W)) * ROWS_PER_W
    pltpu.sync_copy(img_hbm.at[pl.ds(bidx * IMG_COLS_PAD + y0, ROWS_PER_W + 8)],
                    strip)

    for ch in range(4):
        r0 = sid * (WQ_ROWS // 16) + ch * 1312

        pltpu.sync_copy(w_hbm.at[pl.ds(r0, 1312)], stg)

        @plsc.parallel_loop(0, 1312, unroll=4)
        def _(i):
            x = stg[i, pl.ds(0, 16)] * 127.0
            x = x + 0.5 * jnp.sign(x)
            q = jnp.clip(x.astype(jnp.int32), -127, 127)
            stg[i, pl.ds(0, 16)] = q.astype(jnp.float32)

        pltpu.sync_copy(stg, tbl_hbm.at[pl.ds(r0, 1312)])
    plsc.subcore_barrier()

    lane = lax.iota(jnp.int32, 16)
    row_sel = lane // 4
    col_off = lane % 4

    def pair_loop(pr, carry):
        for sub in range(2):
            yl = 2 * pr + sub

            @plsc.parallel_loop(0, 16, unroll=2)
            def _(g):
                x = pl.multiple_of(g * 16, 16)
                a = strip[yl, pl.ds(x, 16)]
                cv = strip[yl + 1, pl.ds(x, 16)]
                sh = jnp.full((16,), x + 1, jnp.int32) + lane
                bv = plsc.load_gather(
                    strip, [jnp.full((16,), yl, jnp.int32), sh])
                dv = plsc.load_gather(
                    strip, [jnp.full((16,), yl + 1, jnp.int32), sh])
                idx = ((a >> 4) * (L * L * L) + (bv >> 4) * (L * L)
                       + (cv >> 4) * L + (dv >> 4))
                xo = pl.ds(pl.multiple_of((g % 8) * 16, 16), 16)
                for m in range(16):
                    idxb[g // 8, m, xo] = idx + OFFS[m]
                frac[0, pl.ds(x, 16)] = (a & 15).astype(jnp.float32) * 0.0625
                frac[1, pl.ds(x, 16)] = (bv & 15).astype(jnp.float32) * 0.0625
                frac[2, pl.ds(x, 16)] = (cv & 15).astype(jnp.float32) * 0.0625
                frac[3, pl.ds(x, 16)] = (dv & 15).astype(jnp.float32) * 0.0625

            copies0 = [pltpu.async_copy(tbl_hbm.at[idxb.at[0, m]],
                                        rows0.at[m], sem0)
                       for m in range(16)]
            copies1 = [pltpu.async_copy(tbl_hbm.at[idxb.at[1, m]],
                                        rows1.at[m], sem1)
                       for m in range(16)]

            def interp(rows, base):
                @plsc.parallel_loop(0, 128, unroll=4)
                def _(p):
                    xg = base + p
                    nf = [plsc.load_gather(
                              frac, [jnp.full((16,), j, jnp.int32),
                                     jnp.full((16,), xg, jnp.int32)])
                          for j in range(4)]
                    v = [rows[m, p, pl.ds(0, 16)] for m in range(16)]
                    for lvl, j in ((8, 3), (4, 2), (2, 1), (1, 0)):
                        v = [v[2 * t] + nf[j] * (v[2 * t + 1] - v[2 * t])
                             for t in range(lvl)]
                    cidx = jnp.full((16,), xg * 4, jnp.int32) + col_off
                    plsc.store_scatter(stag, [row_sel + 4 * sub, cidx], v[0])

            for c in copies0:
                c.wait()
            interp(rows0, 0)
            for c in copies1:
                c.wait()
            interp(rows1, 128)
        pltpu.sync_copy(stag,
                        out_hbm.at[pl.ds((bidx * H_OUT + y0 + 2 * pr) * 4, 8)])
        return carry
    lax.fori_loop(0, ROWS_PER_W // 2, pair_loop, 0)


@functools.cache
def _sc_kernel():
    mesh = plsc.VectorSubcoreMesh(core_axis_name="c", subcore_axis_name="s",
                                  num_cores=2, num_subcores=16)
    return pl.kernel(_sc_body, mesh=mesh, out_type=_sc_out_type(),
                     scratch_types=_sc_scratch_types(),
                     compiler_params=pltpu.CompilerParams(
                         needs_layout_passes=False,
                         use_tc_tiling_on_sc=False))


def kernel(img_in, weight):
    img = img_in.reshape(B, H_IN, H_IN)
    img = jnp.pad(img, ((0, 0), (0, IMG_COLS_PAD - H_IN),
                        (0, IMG_COLS_PAD - H_IN)))
    img = img.reshape(B * IMG_COLS_PAD, IMG_COLS_PAD)
    wpad = jnp.pad(weight, ((0, WQ_ROWS - weight.shape[0]), (0, 0)))
    out, _ = _sc_kernel()(img, wpad)
    return out.reshape(B, 1, H_OUT * 4, H_OUT * 4)


# speedup vs baseline: 1.1720x; 1.1720x over previous
import functools

import jax
import jax.numpy as jnp
from jax import lax
from jax.experimental import pallas as pl
from jax.experimental.pallas import tpu as pltpu
from jax.experimental.pallas import tpu_sc as plsc

L = 17
Q = 16
H_IN = 257
H_OUT = 256
B = 4
GROWS = B * H_OUT
NW = 32
ROWS_PER_W = GROWS // NW
IMG_COLS_PAD = 264
WQ_ROWS = 83968
OFFS = [((m >> 3) & 1) * (L * L * L) + ((m >> 2) & 1) * (L * L)
        + ((m >> 1) & 1) * L + (m & 1) for m in range(16)]

def _quant_body(w_ref, o_ref):
    o_ref[:, :] = jnp.clip(jnp.round(w_ref[:, :] * 127.0), -127.0, 127.0)


def _build_table(wpad):
    flat = wpad.reshape(WQ_ROWS // 8, 128)
    q = pl.pallas_call(
        _quant_body,
        grid=(8,),
        in_specs=[pl.BlockSpec((WQ_ROWS // 64, 128), lambda i: (i, 0))],
        out_specs=pl.BlockSpec((WQ_ROWS // 64, 128), lambda i: (i, 0)),
        out_shape=jax.ShapeDtypeStruct((WQ_ROWS // 8, 128), jnp.float32),
    )(flat)
    return q.reshape(WQ_ROWS, 16)


def _sc_out_type():
    return jax.ShapeDtypeStruct((B * H_OUT * 4, H_OUT * 4), jnp.float32)


def _sc_scratch_types():
    return [
        pltpu.VMEM((ROWS_PER_W + 8, IMG_COLS_PAD), jnp.int32),
        pltpu.VMEM((2, 2, 16, 128), jnp.int32),
        pltpu.VMEM((2048,), jnp.float32),
        pltpu.VMEM((16, 128, 16), jnp.float32),
        pltpu.VMEM((16, 128, 16), jnp.float32),
        pltpu.VMEM((16, 128, 16), jnp.float32),
        pltpu.VMEM((4, 1024), jnp.float32),
        pltpu.SemaphoreType.DMA,
        pltpu.SemaphoreType.DMA,
        pltpu.SemaphoreType.DMA,
    ]


def _sc_body(img_hbm, tbl_hbm, out_hbm,
             strip, idxb, frac, rows0, rows1, rows2, stag,
             sem0, sem1, sem2):
    cid = lax.axis_index("c")
    sid = lax.axis_index("s")
    wid = sid * 2 + cid
    bidx = wid // (H_OUT // ROWS_PER_W)
    y0 = (wid % (H_OUT // ROWS_PER_W)) * ROWS_PER_W
    pltpu.sync_copy(img_hbm.at[pl.ds(bidx * IMG_COLS_PAD + y0, ROWS_PER_W + 8)],
                    strip)

    lane = lax.iota(jnp.int32, 16)
    row_sel = lane // 4
    col_off = lane % 4
    bufs = (rows0, rows1, rows2)
    sems = (sem0, sem1, sem2)

    def pass1(yl):
        par = yl % 2

        @plsc.parallel_loop(0, 16, unroll=2)
        def _(g):
            x = pl.multiple_of(g * 16, 16)
            a = strip[yl, pl.ds(x, 16)]
            cv = strip[yl + 1, pl.ds(x, 16)]
            sh = jnp.full((16,), x + 1, jnp.int32) + lane
            bv = plsc.load_gather(
                strip, [jnp.full((16,), yl, jnp.int32), sh])
            dv = plsc.load_gather(
                strip, [jnp.full((16,), yl + 1, jnp.int32), sh])
            idx = ((a >> 4) * (L * L * L) + (bv >> 4) * (L * L)
                   + (cv >> 4) * L + (dv >> 4))
            xo = pl.ds(pl.multiple_of((g % 8) * 16, 16), 16)
            for m in range(16):
                idxb[par, g // 8, m, xo] = idx + OFFS[m]
            for j, vv in enumerate((a, bv, cv, dv)):
                fo = pl.multiple_of(par * 1024 + j * 256 + x, 16)
                frac[pl.ds(fo, 16)] = (
                    (vv & 15).astype(jnp.float32) * 0.0625)

    def issue(yl, c, b):
        par = yl % 2
        return [pltpu.async_copy(tbl_hbm.at[idxb.at[par, c, m]],
                                 bufs[b].at[m], sems[b])
                for m in range(16)]

    def wait(yl, c, b):
        par = yl % 2
        for m in range(16):
            pltpu.make_async_copy(tbl_hbm.at[idxb.at[par, c, m]],
                                  bufs[b].at[m], sems[b]).wait()

    def interp(yl, c, b):
        par = yl % 2
        rows = bufs[b]

        @plsc.parallel_loop(0, 128, unroll=4)
        def _(p):
            xg = c * 128 + p
            nf = [plsc.load_gather(
                      frac,
                      [jnp.full((16,),
                                par * 1024 + j * 256 + xg, jnp.int32)])
                  for j in range(4)]
            v = [rows[m, p, pl.ds(0, 16)] for m in range(16)]
            for lvl, j in ((8, 3), (4, 2), (2, 1), (1, 0)):
                v = [v[2 * t] + nf[j] * (v[2 * t + 1] - v[2 * t])
                     for t in range(lvl)]
            cidx = jnp.full((16,), xg * 4, jnp.int32) + col_off
            plsc.store_scatter(stag, [row_sel, cidx], v[0])

    def flush(yl):
        pltpu.sync_copy(stag,
                        out_hbm.at[pl.ds((bidx * H_OUT + y0 + yl) * 4, 4)])

    pass1(0)
    issue(0, 0, 0)
    issue(0, 1, 1)

    def triple(t, carry):
        r0 = 3 * t
        pass1(r0 + 1)
        wait(r0, 0, 0)
        interp(r0, 0, 0)
        issue(r0 + 1, 0, 2)
        wait(r0, 1, 1)
        interp(r0, 1, 1)
        issue(r0 + 1, 1, 0)
        flush(r0)
        pass1(r0 + 2)
        wait(r0 + 1, 0, 2)
        interp(r0 + 1, 0, 2)
        issue(r0 + 2, 0, 1)
        wait(r0 + 1, 1, 0)
        interp(r0 + 1, 1, 0)
        issue(r0 + 2, 1, 2)
        flush(r0 + 1)
        pass1(r0 + 3)
        wait(r0 + 2, 0, 1)
        interp(r0 + 2, 0, 1)
        issue(r0 + 3, 0, 0)
        wait(r0 + 2, 1, 2)
        interp(r0 + 2, 1, 2)
        issue(r0 + 3, 1, 1)
        flush(r0 + 2)
        return carry
    lax.fori_loop(0, 10, triple, 0)

    pass1(31)
    wait(30, 0, 0)
    interp(30, 0, 0)
    wait(30, 1, 1)
    interp(30, 1, 1)
    issue(31, 0, 2)
    issue(31, 1, 0)
    flush(30)
    wait(31, 0, 2)
    interp(31, 0, 2)
    wait(31, 1, 0)
    interp(31, 1, 0)
    flush(31)


@functools.cache
def _sc_kernel():
    mesh = plsc.VectorSubcoreMesh(core_axis_name="c", subcore_axis_name="s",
                                  num_cores=2, num_subcores=16)
    return pl.kernel(_sc_body, mesh=mesh, out_type=_sc_out_type(),
                     scratch_types=_sc_scratch_types(),
                     compiler_params=pltpu.CompilerParams(
                         needs_layout_passes=False,
                         use_tc_tiling_on_sc=False))


def kernel(img_in, weight):
    img = img_in.reshape(B, H_IN, H_IN)
    img = jnp.pad(img, ((0, 0), (0, IMG_COLS_PAD - H_IN),
                        (0, IMG_COLS_PAD - H_IN)))
    img = img.reshape(B * IMG_COLS_PAD, IMG_COLS_PAD)
    wpad = jnp.pad(weight, ((0, WQ_ROWS - weight.shape[0]), (0, 0)))
    tbl = _build_table(wpad)
    out = _sc_kernel()(img, tbl)
    return out.reshape(B, 1, H_OUT * 4, H_OUT * 4)


# speedup vs baseline: 1.3423x; 1.1453x over previous
import functools

import jax
import jax.numpy as jnp
from jax import lax
from jax.experimental import pallas as pl
from jax.experimental.pallas import tpu as pltpu
from jax.experimental.pallas import tpu_sc as plsc

L = 17
Q = 16
H_IN = 257
H_OUT = 256
B = 4
GROWS = B * H_OUT
NW = 32
ROWS_PER_W = GROWS // NW
IMG_COLS_PAD = 264
WQ_ROWS = 83968
OFFS = [((m >> 3) & 1) * (L * L * L) + ((m >> 2) & 1) * (L * L)
        + ((m >> 1) & 1) * L + (m & 1) for m in range(16)]

def _quant_body(w_ref, o_ref):
    o_ref[:, :] = jnp.clip(jnp.round(w_ref[:, :] * 127.0), -127.0, 127.0)


def _build_table(wpad):
    flat = wpad.reshape(WQ_ROWS // 8, 128)
    q = pl.pallas_call(
        _quant_body,
        grid=(8,),
        in_specs=[pl.BlockSpec((WQ_ROWS // 64, 128), lambda i: (i, 0))],
        out_specs=pl.BlockSpec((WQ_ROWS // 64, 128), lambda i: (i, 0)),
        out_shape=jax.ShapeDtypeStruct((WQ_ROWS // 8, 128), jnp.float32),
    )(flat)
    return q.reshape(WQ_ROWS, 16)


def _bcast(vec, ii):
    return lax.gather(
        vec, ii[:, None],
        dimension_numbers=lax.GatherDimensionNumbers(
            offset_dims=(), collapsed_slice_dims=(0,), start_index_map=(0,)),
        slice_sizes=(1,),
        mode=lax.GatherScatterMode.PROMISE_IN_BOUNDS)


def _sc_out_type():
    return jax.ShapeDtypeStruct((B * H_OUT * 4, H_OUT * 4), jnp.float32)


def _sc_scratch_types():
    return [
        pltpu.VMEM((ROWS_PER_W + 8, IMG_COLS_PAD), jnp.int32),
        pltpu.VMEM((2, 2, 16, 128), jnp.int32),
        pltpu.VMEM((2048,), jnp.float32),
        pltpu.VMEM((16, 128, 16), jnp.float32),
        pltpu.VMEM((16, 128, 16), jnp.float32),
        pltpu.VMEM((16, 128, 16), jnp.float32),
        pltpu.VMEM((4, 1024), jnp.float32),
        pltpu.SemaphoreType.DMA,
        pltpu.SemaphoreType.DMA,
        pltpu.SemaphoreType.DMA,
    ]


def _sc_body(img_hbm, tbl_hbm, out_hbm,
             strip, idxb, frac, rows0, rows1, rows2, stag,
             sem0, sem1, sem2):
    cid = lax.axis_index("c")
    sid = lax.axis_index("s")
    wid = sid * 2 + cid
    bidx = wid // (H_OUT // ROWS_PER_W)
    y0 = (wid % (H_OUT // ROWS_PER_W)) * ROWS_PER_W
    pltpu.sync_copy(img_hbm.at[pl.ds(bidx * IMG_COLS_PAD + y0, ROWS_PER_W + 8)],
                    strip)

    lane = lax.iota(jnp.int32, 16)
    row_sel = lane // 4
    col_off = lane % 4
    bufs = (rows0, rows1, rows2)
    sems = (sem0, sem1, sem2)

    def pass1(yl):
        par = yl % 2

        @plsc.parallel_loop(0, 16, unroll=2)
        def _(g):
            x = pl.multiple_of(g * 16, 16)
            a = strip[yl, pl.ds(x, 16)]
            cv = strip[yl + 1, pl.ds(x, 16)]
            sh = jnp.full((16,), x + 1, jnp.int32) + lane
            bv = plsc.load_gather(
                strip, [jnp.full((16,), yl, jnp.int32), sh])
            dv = plsc.load_gather(
                strip, [jnp.full((16,), yl + 1, jnp.int32), sh])
            idx = ((a >> 4) * (L * L * L) + (bv >> 4) * (L * L)
                   + (cv >> 4) * L + (dv >> 4))
            xo = pl.ds(pl.multiple_of((g % 8) * 16, 16), 16)
            for m in range(16):
                idxb[par, g // 8, m, xo] = idx + OFFS[m]
            for j, vv in enumerate((a, bv, cv, dv)):
                fo = pl.multiple_of(par * 1024 + j * 256 + x, 16)
                frac[pl.ds(fo, 16)] = (
                    (vv & 15).astype(jnp.float32) * 0.0625)

    def issue(yl, c, b):
        par = yl % 2
        return [pltpu.async_copy(tbl_hbm.at[idxb.at[par, c, m]],
                                 bufs[b].at[m], sems[b])
                for m in range(16)]

    def wait(yl, c, b):
        par = yl % 2
        for m in range(16):
            pltpu.make_async_copy(tbl_hbm.at[idxb.at[par, c, m]],
                                  bufs[b].at[m], sems[b]).wait()

    def interp(yl, c, b):
        par = yl % 2
        rows = bufs[b]

        def grp(g, carry):
            fr = [frac[pl.ds(pl.multiple_of(
                      par * 1024 + j * 256 + c * 128 + g * 16, 16), 16)]
                  for j in range(4)]

            @plsc.parallel_loop(0, 16, unroll=4)
            def _(i):
                p = g * 16 + i
                xg = c * 128 + p
                ii = jnp.full((16,), i, jnp.int32)
                nf = [_bcast(fr[j], ii) for j in range(4)]
                v = [rows[m, p, pl.ds(0, 16)] for m in range(16)]
                for lvl, j in ((8, 3), (4, 2), (2, 1), (1, 0)):
                    v = [v[2 * t] + nf[j] * (v[2 * t + 1] - v[2 * t])
                         for t in range(lvl)]
                cidx = jnp.full((16,), xg * 4, jnp.int32) + col_off
                plsc.store_scatter(stag, [row_sel, cidx], v[0])
            return carry
        lax.fori_loop(0, 8, grp, 0)

    def flush(yl):
        pltpu.sync_copy(stag,
                        out_hbm.at[pl.ds((bidx * H_OUT + y0 + yl) * 4, 4)])

    pass1(0)
    issue(0, 0, 0)
    issue(0, 1, 1)

    def triple(t, carry):
        r0 = 3 * t
        pass1(r0 + 1)
        wait(r0, 0, 0)
        interp(r0, 0, 0)
        issue(r0 + 1, 0, 2)
        wait(r0, 1, 1)
        interp(r0, 1, 1)
        issue(r0 + 1, 1, 0)
        flush(r0)
        pass1(r0 + 2)
        wait(r0 + 1, 0, 2)
        interp(r0 + 1, 0, 2)
        issue(r0 + 2, 0, 1)
        wait(r0 + 1, 1, 0)
        interp(r0 + 1, 1, 0)
        issue(r0 + 2, 1, 2)
        flush(r0 + 1)
        pass1(r0 + 3)
        wait(r0 + 2, 0, 1)
        interp(r0 + 2, 0, 1)
        issue(r0 + 3, 0, 0)
        wait(r0 + 2, 1, 2)
        interp(r0 + 2, 1, 2)
        issue(r0 + 3, 1, 1)
        flush(r0 + 2)
        return carry
    lax.fori_loop(0, 10, triple, 0)

    pass1(31)
    wait(30, 0, 0)
    interp(30, 0, 0)
    wait(30, 1, 1)
    interp(30, 1, 1)
    issue(31, 0, 2)
    issue(31, 1, 0)
    flush(30)
    wait(31, 0, 2)
    interp(31, 0, 2)
    wait(31, 1, 0)
    interp(31, 1, 0)
    flush(31)


@functools.cache
def _sc_kernel():
    mesh = plsc.VectorSubcoreMesh(core_axis_name="c", subcore_axis_name="s",
                                  num_cores=2, num_subcores=16)
    return pl.kernel(_sc_body, mesh=mesh, out_type=_sc_out_type(),
                     scratch_types=_sc_scratch_types(),
                     compiler_params=pltpu.CompilerParams(
                         needs_layout_passes=False,
                         use_tc_tiling_on_sc=False))


def kernel(img_in, weight):
    img = img_in.reshape(B, H_IN, H_IN)
    img = jnp.pad(img, ((0, 0), (0, IMG_COLS_PAD - H_IN),
                        (0, IMG_COLS_PAD - H_IN)))
    img = img.reshape(B * IMG_COLS_PAD, IMG_COLS_PAD)
    wpad = jnp.pad(weight, ((0, WQ_ROWS - weight.shape[0]), (0, 0)))
    tbl = _build_table(wpad)
    out = _sc_kernel()(img, tbl)
    return out.reshape(B, 1, H_OUT * 4, H_OUT * 4)
